# baseline (speedup 1.0000x reference)
import jax
import jax.numpy as jnp
from jax import lax
from jax.experimental import pallas as pl
from jax.experimental.pallas import tpu as pltpu
from jax.experimental.pallas import tpu_sc as plsc

_NC = 2
_NS = 16
_NW = _NC * _NS
_LANES = 16


def _make_sc_kernel(Bsz, Lsz, V, D):
    lt = Lsz // 8
    bt = Bsz // 128
    dh = D // 8

    mesh = plsc.VectorSubcoreMesh(core_axis_name="c", subcore_axis_name="s")

    def body(x4_hbm, tab_hbm, pos_hbm, out_hbm,
             idx_v, pos_v, rbuf_a, rbuf_b, tbuf_a, tbuf_b,
             gsem_a, gsem_b, osem_a, osem_b):
        c = lax.axis_index("c")
        s = lax.axis_index("s")
        wid = s * _NC + c
        pltpu.sync_copy(x4_hbm.at[:, wid], idx_v)
        pltpu.sync_copy(pos_hbm, pos_v)

        def fire_gather(rbuf, sem, l):
            return pltpu.async_copy(
                tab_hbm.at[idx_v.at[l // 8, l % 8]], rbuf, sem
            )

        def transpose_add(l, rbuf, tbuf):
            lvec = jnp.full((_LANES,), l, jnp.int32)
            base_rows = lax.iota(jnp.int32, _LANES)

            @plsc.parallel_loop(0, D, unroll=2)
            def _(d):
                dvec = jnp.full((_LANES,), d, jnp.int32)
                pvals = plsc.load_gather(pos_v, [lvec, dvec])
                dhi = d // 8
                dli = lax.rem(d, 8)
                for bg in range(8):
                    rows = base_rows + (bg * _LANES)
                    vals = plsc.load_gather(rbuf, [rows, dvec])
                    tbuf[dhi, dli, pl.ds(bg * _LANES, _LANES)] = vals + pvals

        def fire_out(tbuf, sem, l):
            return pltpu.async_copy(tbuf, out_hbm.at[l, :, wid], sem)

        def pair_body(g, carry):
            l_a = g * 2
            l_b = l_a + 1
            h_a = fire_gather(rbuf_a, gsem_a, l_a)
            h_b = fire_gather(rbuf_b, gsem_b, l_b)
            h_a.wait()
            transpose_add(l_a, rbuf_a, tbuf_a)
            o_a = fire_out(tbuf_a, osem_a, l_a)
            h_b.wait()
            transpose_add(l_b, rbuf_b, tbuf_b)
            o_b = fire_out(tbuf_b, osem_b, l_b)
            o_a.wait()
            o_b.wait()
            return carry

        lax.fori_loop(0, Lsz // 2, pair_body, None)

    return pl.kernel(
        body,
        out_type=jax.ShapeDtypeStruct((Lsz, dh, bt, 8, 128), jnp.float32),
        mesh=mesh,
        compiler_params=pltpu.CompilerParams(
            use_tc_tiling_on_sc=False, needs_layout_passes=False),
        scratch_types=[
            pltpu.VMEM((lt, 8, 128), jnp.int32),
            pltpu.VMEM((Lsz, D), jnp.float32),
            pltpu.VMEM((128, D), jnp.float32),
            pltpu.VMEM((128, D), jnp.float32),
            pltpu.VMEM((dh, 8, 128), jnp.float32),
            pltpu.VMEM((dh, 8, 128), jnp.float32),
            pltpu.SemaphoreType.DMA,
            pltpu.SemaphoreType.DMA,
            pltpu.SemaphoreType.DMA,
            pltpu.SemaphoreType.DMA,
        ],
    )


def kernel(x, word_table, pos_table):
    Bsz, Lsz = x.shape
    V, D = word_table.shape
    lt = Lsz // 8
    bt = Bsz // 128

    x4 = jnp.transpose(x).reshape(lt, 8, bt, 128).transpose(0, 2, 1, 3)
    pos_rows = pos_table[1 : Lsz + 1]

    sc = _make_sc_kernel(Bsz, Lsz, V, D)
    out5 = sc(x4, word_table, pos_rows)
    return jnp.transpose(out5, (2, 4, 0, 1, 3)).reshape(Bsz, Lsz, D)
